# Initial kernel scaffold; baseline (speedup 1.0000x reference)
import jax
import jax.numpy as jnp
from jax.experimental import pallas as pl


def kernel(features, edge_list, W, b, a, W_out, b_out, a_out):
    raise NotImplementedError("write your pallas kernel here")


# speedup vs baseline: 10.3668x; 10.3668x over previous
import functools

import jax
import jax.numpy as jnp
from jax import lax
from jax.experimental import pallas as pl
from jax.experimental.pallas import tpu as pltpu
from jax.experimental.pallas import tpu_sc as plsc

_ALPHA = 0.2
_C = 128
_L = 16


def _lrelu_exp(x):
    return jnp.exp(jnp.maximum(x, _ALPHA * x))


def _bfperm(t):
    M = t.shape[0]
    return (t.reshape(M, 4, 2, _L).transpose(0, 1, 3, 2).reshape(M, 128)
            .astype(jnp.bfloat16))


def _tc_embed(features, Wc, bc, As, N1A):
    N, D = features.shape
    H2 = Wc.shape[1]
    BR = 1000

    def body(x_ref, wc_ref, bc_ref, as_ref, h_ref, s_ref):
        h = jnp.dot(x_ref[...], wc_ref[...], preferred_element_type=jnp.float32)
        h = h + bc_ref[...]
        for p in range(4):
            h_ref[p] = h[:, 128 * p:128 * (p + 1)]
        s_ref[...] = jnp.dot(h, as_ref[...], preferred_element_type=jnp.float32)

    return pl.pallas_call(
        body,
        grid=(N // BR,),
        in_specs=[
            pl.BlockSpec((BR, D), lambda i: (i, 0)),
            pl.BlockSpec((D, H2), lambda i: (0, 0)),
            pl.BlockSpec((1, H2), lambda i: (0, 0)),
            pl.BlockSpec((H2, 128), lambda i: (0, 0)),
        ],
        out_specs=[
            pl.BlockSpec((4, BR, 128), lambda i: (0, i, 0)),
            pl.BlockSpec((BR, 128), lambda i: (i, 0)),
        ],
        out_shape=[
            jax.ShapeDtypeStruct((4, N1A, 128), jnp.float32),
            jax.ShapeDtypeStruct((N, 128), jnp.float32),
        ],
    )(features, Wc, bc, As)


def _tc_rsum(rs_parts):
    _, NT, RS = rs_parts.shape

    def body(a_ref, o_ref):
        o_ref[...] = jnp.sum(a_ref[...], axis=1)

    return pl.pallas_call(
        body,
        grid=(1,),
        in_specs=[pl.BlockSpec((4, NT, RS), lambda p: (0, 0, 0))],
        out_specs=pl.BlockSpec((4, RS), lambda p: (0, 0)),
        out_shape=jax.ShapeDtypeStruct((4, RS), jnp.float32),
    )(rs_parts)


def _tc_mid(acc1, rs3, WoT, boP, AoM, N, NC, N1A):
    BR = 1000

    def body(a_ref, rs_ref, wo_ref, bo_ref, ao_ref, h2_ref, s_ref):
        a = a_ref[...]
        rs = rs_ref[...]
        cols = []
        for p in range(4):
            for j in range(2):
                hp = a[p, :, j * 64:(j + 1) * 64]
                v = hp / rs[p, :, j:j + 1]
                cols.append(jnp.where(v > 0, v, jnp.exp(jnp.minimum(v, 0.0)) - 1.0))
        x2 = jnp.concatenate(cols, axis=1)
        h2 = jnp.dot(x2, wo_ref[...], preferred_element_type=jnp.float32)
        h2 = h2 + bo_ref[...]
        col = lax.broadcasted_iota(jnp.int32, (BR, 128), 1)
        h2 = jnp.where(col == NC, 1.0, h2)
        h2_ref[...] = h2
        s_ref[...] = jnp.dot(h2, ao_ref[...], preferred_element_type=jnp.float32)

    return pl.pallas_call(
        body,
        grid=(N // BR,),
        in_specs=[
            pl.BlockSpec((4, BR, 128), lambda i: (0, i, 0)),
            pl.BlockSpec((4, BR, 2), lambda i: (0, i, 0)),
            pl.BlockSpec((512, 128), lambda i: (0, 0)),
            pl.BlockSpec((1, 128), lambda i: (0, 0)),
            pl.BlockSpec((128, 128), lambda i: (0, 0)),
        ],
        out_specs=[
            pl.BlockSpec((BR, 128), lambda i: (i, 0)),
            pl.BlockSpec((BR, 128), lambda i: (i, 0)),
        ],
        out_shape=[
            jax.ShapeDtypeStruct((N1A, 128), jnp.float32),
            jax.ShapeDtypeStruct((N, 128), jnp.float32),
        ],
    )(acc1, rs3, WoT, boP, AoM)


def _tc_final(acc2, N, NC):
    BR = 1000

    def body(a_ref, o_ref):
        a = a_ref[0] + a_ref[1]
        logit = a[:, :NC] / a[:, NC:NC + 1]
        m = jnp.max(logit, axis=1, keepdims=True)
        lz = logit - m
        lse = jnp.log(jnp.sum(jnp.exp(lz), axis=1, keepdims=True))
        o_ref[...] = jnp.concatenate(
            [lz - lse, jnp.zeros((BR, 128 - NC), jnp.float32)], axis=1)

    return pl.pallas_call(
        body,
        grid=(N // BR,),
        in_specs=[pl.BlockSpec((2, BR, 128), lambda i: (0, i, 0))],
        out_specs=pl.BlockSpec((BR, 128), lambda i: (i, 0)),
        out_shape=jax.ShapeDtypeStruct((N, 128), jnp.float32),
    )(acc2)


_G = 8


def _sc_edge_e(s1t, s2t, srcm, dstm, N1S, CH):
    RS = 2 * N1S
    NG = CH // _G
    mesh = plsc.VectorSubcoreMesh(core_axis_name="c", subcore_axis_name="s")

    @functools.partial(
        pl.kernel,
        out_type=(jax.ShapeDtypeStruct((4, 16, CH, 2 * _C), jnp.float32),
                  jax.ShapeDtypeStruct((4, 16, RS), jnp.float32)),
        mesh=mesh,
        compiler_params=pltpu.CompilerParams(needs_layout_passes=False),
        scratch_types=[
            pltpu.VMEM((2 * N1S,), jnp.float32),
            pltpu.VMEM((2 * N1S,), jnp.float32),
            pltpu.VMEM((RS,), jnp.float32),
            pltpu.VMEM((_G, _C), jnp.int32),
            pltpu.VMEM((_G, _C), jnp.int32),
            pltpu.VMEM((_G, 2 * _C), jnp.float32),
        ],
    )
    def k(s1_hbm, s2_hbm, srcm_hbm, dstm_hbm, e_hbm, rs_hbm,
          s1v, s2v, rsv, srcg, dstg, evg):
        cid = lax.axis_index("c")
        sid = lax.axis_index("s")
        zf = jnp.zeros((_L,), jnp.float32)
        m1 = lax.iota(jnp.int32, _L) < 1

        def zero_rsv():
            def b(r, carry):
                rsv[pl.ds(r * _L, _L)] = zf
                return carry
            lax.fori_loop(0, RS // _L, b, 0)

        for pp in range(2):
            p = 2 * cid + pp
            zero_rsv()
            pltpu.sync_copy(s1_hbm.at[p], s1v)
            pltpu.sync_copy(s2_hbm.at[p], s2v)

            def group(g, carry):
                pltpu.sync_copy(srcm_hbm.at[sid, pl.ds(g * _G, _G)], srcg)
                pltpu.sync_copy(dstm_hbm.at[sid, pl.ds(g * _G, _G)], dstg)

                def chunk(c, ccarry):
                    def grp(i, icarry):
                        sv = srcg[c, pl.ds(i * _L, _L)]
                        dv = dstg[c, pl.ds(i * _L, _L)]
                        l0 = (plsc.load_gather(s1v, [sv])
                              + plsc.load_gather(s2v, [dv]))
                        l1 = (plsc.load_gather(s1v, [sv + N1S])
                              + plsc.load_gather(s2v, [dv + N1S]))
                        e0v = _lrelu_exp(l0)
                        e1v = _lrelu_exp(l1)
                        evg[c, pl.ds(i * _L, _L)] = e0v
                        evg[c, pl.ds(_C + i * _L, _L)] = e1v
                        sv2 = sv * 2
                        for l in range(_L):
                            lsel = jnp.full((_L,), l, jnp.int32)
                            plsc.addupdate_scatter(rsv, [sv2[lsel]],
                                                   e0v[lsel], mask=m1)
                            plsc.addupdate_scatter(rsv, [sv2[lsel] + 1],
                                                   e1v[lsel], mask=m1)
                        return icarry
                    lax.fori_loop(0, _C // _L, grp, 0)
                    return ccarry
                lax.fori_loop(0, _G, chunk, 0)
                pltpu.sync_copy(evg, e_hbm.at[p, sid, pl.ds(g * _G, _G)])
                return carry
            lax.fori_loop(0, NG, group, 0)
            pltpu.sync_copy(rsv, rs_hbm.at[p, sid])

    return k(s1t, s2t, srcm, dstm)


def _sc_edge_e2(s1t2, s2t2, srcm, dstm, SL):
    CH = srcm.shape[1]
    NG = CH // _G
    mesh = plsc.VectorSubcoreMesh(core_axis_name="c", subcore_axis_name="s")

    @functools.partial(
        pl.kernel,
        out_type=jax.ShapeDtypeStruct((32, CH, _C), jnp.float32),
        mesh=mesh,
        compiler_params=pltpu.CompilerParams(needs_layout_passes=False),
        scratch_types=[
            pltpu.VMEM((SL,), jnp.float32),
            pltpu.VMEM((SL,), jnp.float32),
            pltpu.VMEM((_G, _C), jnp.int32),
            pltpu.VMEM((_G, _C), jnp.int32),
            pltpu.VMEM((_G, _C), jnp.float32),
        ],
    )
    def k(s1_hbm, s2_hbm, srcm_hbm, dstm_hbm, e_hbm,
          s1v, s2v, srcg, dstg, evg):
        cid = lax.axis_index("c")
        sid = lax.axis_index("s")
        wid = cid * 16 + sid
        pltpu.sync_copy(s1_hbm, s1v)
        pltpu.sync_copy(s2_hbm, s2v)

        def group(g, carry):
            pltpu.sync_copy(srcm_hbm.at[wid, pl.ds(g * _G, _G)], srcg)
            pltpu.sync_copy(dstm_hbm.at[wid, pl.ds(g * _G, _G)], dstg)

            def chunk(c, ccarry):
                def grp(i, icarry):
                    sv = srcg[c, pl.ds(i * _L, _L)]
                    dv = dstg[c, pl.ds(i * _L, _L)]
                    l0 = plsc.load_gather(s1v, [sv]) + plsc.load_gather(s2v, [dv])
                    evg[c, pl.ds(i * _L, _L)] = _lrelu_exp(l0)
                    return icarry
                lax.fori_loop(0, _C // _L, grp, 0)
                return ccarry
            lax.fori_loop(0, _G, chunk, 0)
            pltpu.sync_copy(evg, e_hbm.at[wid, pl.ds(g * _G, _G)])
            return carry
        lax.fori_loop(0, NG, group, 0)

    return k(s1t2, s2t2, srcm, dstm)


def _scatter_pipeline(tab_hbm, e_hbm_slice_fn, srcm_hbm, dstm_hbm, acc,
                      srcg, dstg, eg, rows0, rows1, sg0, sg1, ss0, ss1,
                      st0, st1, st2, sid_idx, poff, CH, nsc):
    NG = CH // _G

    def group(g, carry):
        cd_src = pltpu.async_copy(srcm_hbm.at[sid_idx, pl.ds(g * _G, _G)],
                                  srcg, st0)
        cd_dst = pltpu.async_copy(dstm_hbm.at[sid_idx, pl.ds(g * _G, _G)],
                                  dstg, st1)
        cd_e = pltpu.async_copy(e_hbm_slice_fn(g), eg, st2)
        cd_dst.wait()

        if poff is not None:
            def ob(r, ocarry):
                for q in range(_C // _L):
                    dstg[r, q * _L:(q + 1) * _L] = (
                        dstg[r, q * _L:(q + 1) * _L] + poff)
                return ocarry
            lax.fori_loop(0, _G, ob, 0)

        bufs = (rows0, rows1)
        sgs = (sg0, sg1)
        sss = (ss0, ss1)
        cg = [None] * _G
        cs = [None] * _G
        cg[0] = pltpu.async_copy(tab_hbm.at[dstg.at[0]], rows0, sg0)
        cd_e.wait()
        cd_src.wait()
        for c in range(_G):
            b = c % 2
            if c + 1 < _G:
                if c >= 1:
                    cs[c - 1].wait()
                cg[c + 1] = pltpu.async_copy(
                    tab_hbm.at[dstg.at[c + 1]], bufs[1 - b], sgs[1 - b])
            cg[c].wait()
            rows = bufs[b]

            def grp(i, icarry):
                e0v = eg[c, pl.ds(i * _L, _L)]
                if nsc == 8:
                    e1v = eg[c, pl.ds(_C + i * _L, _L)]
                for l in range(_L):
                    j = i * _L + l
                    lsel = jnp.full((_L,), l, jnp.int32)
                    e0 = e0v[lsel]
                    e1 = e1v[lsel] if nsc == 8 else e0
                    for q in range(nsc if nsc != 8 else 4):
                        rows[j, q * _L:(q + 1) * _L] = (
                            rows[j, q * _L:(q + 1) * _L] * e0)
                    if nsc == 8:
                        for q in range(4, 8):
                            rows[j, q * _L:(q + 1) * _L] = (
                                rows[j, q * _L:(q + 1) * _L] * e1)
                return icarry
            lax.fori_loop(0, _C // _L, grp, 0)

            cs[c] = pltpu.async_copy(rows, acc.at[srcg.at[c]], sss[b], add=True)
        cs[_G - 2].wait()
        cs[_G - 1].wait()
        return carry
    lax.fori_loop(0, NG, group, 0)


def _sc_scatter1(tab1, e_rec, srcm, dstm, N1, NACC):
    CH = srcm.shape[1]
    ZR = NACC // 16
    NF, REM = ZR // _C, ZR % _C
    mesh = plsc.VectorSubcoreMesh(core_axis_name="c", subcore_axis_name="s")

    @functools.partial(
        pl.kernel,
        out_type=jax.ShapeDtypeStruct((4, NACC, 128), jnp.float32),
        mesh=mesh,
        compiler_params=pltpu.CompilerParams(needs_layout_passes=False),
        scratch_types=[
            pltpu.VMEM_SHARED((NACC, 128), jnp.float32),
            pltpu.VMEM((_G, _C), jnp.int32),
            pltpu.VMEM((_G, _C), jnp.int32),
            pltpu.VMEM((_G, 2 * _C), jnp.float32),
            pltpu.VMEM((_C, 128), jnp.float32),
            pltpu.VMEM((_C, 128), jnp.float32),
            pltpu.SemaphoreType.DMA,
            pltpu.SemaphoreType.DMA,
            pltpu.SemaphoreType.DMA,
            pltpu.SemaphoreType.DMA,
            pltpu.SemaphoreType.DMA,
            pltpu.SemaphoreType.DMA,
            pltpu.SemaphoreType.DMA,
        ],
    )
    def k(tab_hbm, e_hbm, srcm_hbm, dstm_hbm, acc_hbm,
          acc, srcg, dstg, eg, rows0, rows1, sg0, sg1, ss0, ss1,
          st0, st1, st2):
        cid = lax.axis_index("c")
        sid = lax.axis_index("s")
        zf = jnp.zeros((_L,), jnp.float32)

        def zero_rows():
            def b(r, carry):
                for q in range(128 // _L):
                    rows0[r, q * _L:(q + 1) * _L] = zf
                    rows1[r, q * _L:(q + 1) * _L] = zf
                return carry
            lax.fori_loop(0, _C, b, 0)

        base = sid * ZR

        def zero_acc():
            for kk in range(NF):
                pltpu.sync_copy(rows0, acc.at[pl.ds(base + kk * _C, _C)])
            if REM:
                pltpu.sync_copy(rows0.at[0:REM],
                                acc.at[pl.ds(base + NF * _C, REM)])

        for pp in range(2):
            p = 2 * cid + pp
            zero_rows()
            zero_acc()
            plsc.subcore_barrier()
            poff = p * N1

            _scatter_pipeline(
                tab_hbm, lambda g: e_hbm.at[p, sid, pl.ds(g * _G, _G)],
                srcm_hbm, dstm_hbm, acc, srcg, dstg, eg, rows0, rows1,
                sg0, sg1, ss0, ss1, st0, st1, st2, sid, poff, CH, nsc=8)

            plsc.subcore_barrier()
            for kk in range(NF):
                pltpu.sync_copy(acc.at[pl.ds(base + kk * _C, _C)],
                                acc_hbm.at[p, pl.ds(base + kk * _C, _C)])
            if REM:
                pltpu.sync_copy(acc.at[pl.ds(base + NF * _C, REM)],
                                acc_hbm.at[p, pl.ds(base + NF * _C, REM)])
            plsc.subcore_barrier()

    return k(tab1, e_rec, srcm, dstm)


def _sc_scatter2(tab2, e_rec2, srcm, dstm, NACC):
    CH = srcm.shape[1]
    ZR = NACC // 16
    NF, REM = ZR // _C, ZR % _C
    mesh = plsc.VectorSubcoreMesh(core_axis_name="c", subcore_axis_name="s")

    @functools.partial(
        pl.kernel,
        out_type=jax.ShapeDtypeStruct((2, NACC, 128), jnp.float32),
        mesh=mesh,
        compiler_params=pltpu.CompilerParams(needs_layout_passes=False),
        scratch_types=[
            pltpu.VMEM_SHARED((NACC, 128), jnp.float32),
            pltpu.VMEM((_G, _C), jnp.int32),
            pltpu.VMEM((_G, _C), jnp.int32),
            pltpu.VMEM((_G, _C), jnp.float32),
            pltpu.VMEM((_C, 128), jnp.float32),
            pltpu.VMEM((_C, 128), jnp.float32),
            pltpu.SemaphoreType.DMA,
            pltpu.SemaphoreType.DMA,
            pltpu.SemaphoreType.DMA,
            pltpu.SemaphoreType.DMA,
            pltpu.SemaphoreType.DMA,
            pltpu.SemaphoreType.DMA,
            pltpu.SemaphoreType.DMA,
        ],
    )
    def k(tab_hbm, e_hbm, srcm_hbm, dstm_hbm, out_hbm,
          acc, srcg, dstg, eg, rows0, rows1, sg0, sg1, ss0, ss1,
          st0, st1, st2):
        cid = lax.axis_index("c")
        sid = lax.axis_index("s")
        wid = cid * 16 + sid
        zf = jnp.zeros((_L,), jnp.float32)

        def zero_rows():
            def b(r, carry):
                for q in range(128 // _L):
                    rows0[r, q * _L:(q + 1) * _L] = zf
                    rows1[r, q * _L:(q + 1) * _L] = zf
                return carry
            lax.fori_loop(0, _C, b, 0)

        base = sid * ZR
        zero_rows()
        for kk in range(NF):
            pltpu.sync_copy(rows0, acc.at[pl.ds(base + kk * _C, _C)])
        if REM:
            pltpu.sync_copy(rows0.at[0:REM], acc.at[pl.ds(base + NF * _C, REM)])
        plsc.subcore_barrier()

        _scatter_pipeline(
            tab_hbm, lambda g: e_hbm.at[wid, pl.ds(g * _G, _G)],
            srcm_hbm, dstm_hbm, acc, srcg, dstg, eg, rows0, rows1,
            sg0, sg1, ss0, ss1, st0, st1, st2, wid, None, CH, nsc=3)

        plsc.subcore_barrier()
        for kk in range(NF):
            pltpu.sync_copy(acc.at[pl.ds(base + kk * _C, _C)],
                            out_hbm.at[cid, pl.ds(base + kk * _C, _C)])
        if REM:
            pltpu.sync_copy(acc.at[pl.ds(base + NF * _C, REM)],
                            out_hbm.at[cid, pl.ds(base + NF * _C, REM)])

    return k(tab2, e_rec2, srcm, dstm)


def kernel(features, edge_list, W, b, a, W_out, b_out, a_out):
    N, D = features.shape
    NH, HID, _ = W.shape
    NC = W_out.shape[0]
    E = edge_list.shape[1]
    N1 = N + 1
    N1S = ((N1 + 63) // 64) * 64
    NACC = ((N1 + 127) // 128) * 128
    H2 = NH * HID
    f32 = jnp.float32

    Wc = jnp.transpose(W, (2, 0, 1)).reshape(D, H2)
    bc = b.reshape(1, H2)
    a1 = a[:, 0, :HID].reshape(H2)
    a2 = a[:, 0, HID:].reshape(H2)
    r512 = jnp.arange(H2)
    As1 = jnp.zeros((H2, NH), f32).at[r512, r512 // HID].set(a1)
    As2 = jnp.zeros((H2, NH), f32).at[r512, r512 // HID].set(a2)
    As = jnp.concatenate([As1, As2, jnp.zeros((H2, 128 - 2 * NH), f32)], axis=1)
    N1A = ((N1 + 7) // 8) * 8
    htab, s = _tc_embed(features, Wc, bc, As, N1A)

    tab1 = htab.reshape(4 * N1A, 128)
    sT = jnp.pad(s[:, :2 * NH].T, ((0, 0), (0, N1S - N)))
    s1t = sT[:NH].reshape(4, 2 * N1S)
    s2t = sT[NH:].reshape(4, 2 * N1S)

    EP = ((E + 16 * _C * _G - 1) // (16 * _C * _G)) * (16 * _C * _G)
    pad = jnp.full((EP - E,), N, jnp.int32)
    srcp = jnp.concatenate([edge_list[0].astype(jnp.int32), pad])
    dstp = jnp.concatenate([edge_list[1].astype(jnp.int32), pad])
    CH1 = EP // (16 * _C)
    CH2 = EP // (32 * _C)
    srcm1 = srcp.reshape(16, CH1, _C)
    dstm1 = dstp.reshape(16, CH1, _C)
    srcm2 = srcp.reshape(32, CH2, _C)
    dstm2 = dstp.reshape(32, CH2, _C)

    e_rec, rs_parts = _sc_edge_e(s1t, s2t, srcm1, dstm1, N1S, CH1)
    acc1 = _sc_scatter1(tab1, e_rec, srcm1, dstm1, N1A, NACC)
    rs = _tc_rsum(rs_parts)
    rs3 = rs.reshape(4, N1S, 2)

    WoT = jnp.pad(W_out.T, ((0, 0), (0, 128 - NC)))
    boP = jnp.pad(b_out, (0, 128 - NC)).reshape(1, 128)
    AoM = jnp.stack(
        [jnp.pad(a_out[0, :NC], (0, 128 - NC)),
         jnp.pad(a_out[0, NC:], (0, 128 - NC))], axis=1)
    AoM = jnp.pad(AoM, ((0, 0), (0, 126)))
    tab2, sp = _tc_mid(acc1, rs3, WoT, boP, AoM, N, NC, N1A)

    SL = ((N1 + 7) // 8) * 8
    s1t2 = jnp.pad(sp[:, 0], (0, SL - N))
    s2t2 = jnp.pad(sp[:, 1], (0, SL - N))

    e_rec2 = _sc_edge_e2(s1t2, s2t2, srcm2, dstm2, SL)
    acc2 = _sc_scatter2(tab2, e_rec2, srcm2, dstm2, NACC)

    out = _tc_final(acc2, N, NC)
    return out[:, :NC]
